# Initial kernel scaffold; baseline (speedup 1.0000x reference)
import jax
import jax.numpy as jnp
from jax.experimental import pallas as pl


def kernel(x, t, edge_index, W_x, W_t, Q_alpha_w, Q_alpha_b, K_alpha_w, K_alpha_b, Q_beta_w, Q_beta_b, K_beta_w, K_beta_b):
    raise NotImplementedError("write your pallas kernel here")


# speedup vs baseline: 2.7356x; 2.7356x over previous
import functools

import jax
import jax.numpy as jnp
from jax.experimental import pallas as pl


def _node_tables_body(x_ref, t_ref, wx_ref, wt_ref, qaw_ref, qab_ref,
                      kaw_ref, kab_ref, qbw_ref, qbb_ref, kbw_ref, kbb_ref,
                      qb_ref, cmx_ref, qa_ref, cmt_ref):
    x = x_ref[...]
    t = t_ref[...]
    d = x.shape[1]
    scale = 1.0 / (d ** 0.5)
    f32 = jnp.float32
    qb = jax.lax.dot_general(x, qbw_ref[...], (((1,), (1,)), ((), ())),
                             preferred_element_type=f32) + qbb_ref[...][None, :]
    qb_ref[...] = qb * scale
    kb = jax.lax.dot_general(x, kbw_ref[...], (((1,), (1,)), ((), ())),
                             preferred_element_type=f32) + kbb_ref[...][None, :]
    mx = jax.lax.dot_general(x, wx_ref[...], (((1,), (1,)), ((), ())),
                             preferred_element_type=f32)
    cmx_ref[...] = jnp.concatenate([kb, mx], axis=1)
    qa = jax.lax.dot_general(t, qaw_ref[...], (((1,), (1,)), ((), ())),
                             preferred_element_type=f32) + qab_ref[...][None, :]
    qa_ref[...] = qa * scale
    ka = jax.lax.dot_general(t, kaw_ref[...], (((1,), (1,)), ((), ())),
                             preferred_element_type=f32) + kab_ref[...][None, :]
    mt = jax.lax.dot_general(t, wt_ref[...], (((1,), (1,)), ((), ())),
                             preferred_element_type=f32)
    cmt_ref[...] = jnp.concatenate([ka, mt], axis=1)


def _node_tables(x, t, W_x, W_t, Qa_w, Qa_b, Ka_w, Ka_b, Qb_w, Qb_b, Kb_w, Kb_b):
    n, d = x.shape
    bn = 2000
    grid = (n // bn,)
    row_spec = pl.BlockSpec((bn, d), lambda i: (i, 0))
    w_spec = pl.BlockSpec((d, d), lambda i: (0, 0))
    b_spec = pl.BlockSpec((d,), lambda i: (0,))
    out_row = pl.BlockSpec((bn, d), lambda i: (i, 0))
    out_row2 = pl.BlockSpec((bn, 2 * d), lambda i: (i, 0))
    f32 = jnp.float32
    return pl.pallas_call(
        _node_tables_body,
        grid=grid,
        in_specs=[row_spec, row_spec, w_spec, w_spec, w_spec, b_spec,
                  w_spec, b_spec, w_spec, b_spec, w_spec, b_spec],
        out_specs=[out_row, out_row2, out_row, out_row2],
        out_shape=[
            jax.ShapeDtypeStruct((n, d), f32),
            jax.ShapeDtypeStruct((n, 2 * d), f32),
            jax.ShapeDtypeStruct((n, d), f32),
            jax.ShapeDtypeStruct((n, 2 * d), f32),
        ],
    )(x, t, W_x, W_t, Qa_w, Qa_b, Ka_w, Ka_b, Qb_w, Qb_b, Kb_w, Kb_b)


def kernel(x, t, edge_index, W_x, W_t, Q_alpha_w, Q_alpha_b, K_alpha_w,
           K_alpha_b, Q_beta_w, Q_beta_b, K_beta_w, K_beta_b):
    n, d = x.shape
    row = edge_index[0]
    col = edge_index[1]
    qb, cmx, qa, cmt = _node_tables(x, t, W_x, W_t, Q_alpha_w, Q_alpha_b,
                                    K_alpha_w, K_alpha_b, Q_beta_w, Q_beta_b,
                                    K_beta_w, K_beta_b)
    def side(q_tab, km_tab):
        q_e = q_tab[row]
        km_e = km_tab[col]
        s = jnp.sum(q_e * km_e[:, :d], axis=-1)
        w = jnp.exp(s)
        wsum = jax.ops.segment_sum(w, row, num_segments=n)
        aggr = jax.ops.segment_sum(w[:, None] * km_e[:, d:], row, num_segments=n)
        return aggr / jnp.where(wsum == 0.0, 1.0, wsum)[:, None]

    out_x = side(qb, cmx)
    out_t = side(qa, cmt)
    return out_x, out_t


# speedup vs baseline: 6.2255x; 2.2757x over previous
import jax
import jax.numpy as jnp
from jax import lax
from jax.experimental import pallas as pl
from jax.experimental.pallas import tpu as pltpu
from jax.experimental.pallas import tpu_sc as plsc

_F32 = jnp.float32
_I32 = jnp.int32


def _tables_body(xt_ref, qw_ref, qb_ref, kw_ref, kb_ref, ww_ref,
                 q_out, k_out, m_out):
    xb = xt_ref[...]
    d = xb.shape[1]
    scale = 1.0 / (d ** 0.5)
    dn = (((1,), (1,)), ((), ()))
    q = lax.dot_general(xb, qw_ref[0], dn, preferred_element_type=_F32)
    q_out[...] = (q + qb_ref[0]) * scale
    k = lax.dot_general(xb, kw_ref[0], dn, preferred_element_type=_F32)
    k_out[...] = k + kb_ref[0]
    m_out[...] = lax.dot_general(xb, ww_ref[0], dn,
                                 preferred_element_type=_F32)


def _tables(xt, qw, qb, kw, kb, ww):
    n2, d = xt.shape
    bn = 2000
    nb = (n2 // 2) // bn
    grid = (2, nb)
    row = pl.BlockSpec((bn, d), lambda i, j: (i * nb + j, 0))
    w_s = pl.BlockSpec((1, d, d), lambda i, j: (i, 0, 0))
    b_s = pl.BlockSpec((1, 1, d), lambda i, j: (i, 0, 0))
    return pl.pallas_call(
        _tables_body,
        grid=grid,
        in_specs=[row, w_s, b_s, w_s, b_s, w_s],
        out_specs=[row, row, row],
        out_shape=[jax.ShapeDtypeStruct((n2, d), _F32)] * 3,
    )(xt, qw, qb, kw, kb, ww)


_NT = 16
_SUP = 2000
_C = 80
_CPS = _SUP // _C
_G = 16
_HB = 4992
_SLAB = 320
_ACCR = 5120
_DUMP = 5088
_WPAD = 10240


def _edge_body(q_tab, k_tab, m_tab, erow, ecol, out_xt,
               row_f, col_f, radj, cadj, row2d, idxb,
               bufa, bufb, wfull, wloc, wred, dchunk, obuf,
               acc, wstage, s1, s2):
    n = out_xt.shape[0] // 2
    d = out_xt.shape[1]
    e = erow.shape[0]
    ept = e // _NT
    nsup = ept // _SUP
    c = lax.axis_index("c")
    s = lax.axis_index("s")
    cn = c * n
    ebase = s * ept
    lane = lax.iota(_I32, 16)

    def zw(k, _):
        wloc[pl.ds(k * 16, 16)] = jnp.zeros((16,), _F32)
        return 0
    lax.fori_loop(0, _WPAD // 16, zw, 0)

    def load_super(si):
        base = ebase + si * _SUP
        pltpu.sync_copy(erow.at[pl.ds(base, _SUP)], row_f)
        pltpu.sync_copy(ecol.at[pl.ds(base, _SUP)], col_f)

        def adj(k, _):
            rv = row_f[pl.ds(k * 16, 16)]
            cv = col_f[pl.ds(k * 16, 16)]
            radj[pl.ds(k * 16, 16)] = rv + cn
            cadj[pl.ds(k * 16, 16)] = cv + cn
            ci = k // (_C // 16)
            off = (k % (_C // 16)) * 16
            row2d[ci, pl.ds(off, 16)] = rv
            return 0
        lax.fori_loop(0, _SUP // 16, adj, 0)

    def super_a(si, _):
        load_super(si)

        def chunk_a(ci, _):
            off = ci * _C
            cq = pltpu.async_copy(q_tab.at[radj.at[pl.ds(off, _C)]], bufa, s1)
            ck = pltpu.async_copy(k_tab.at[cadj.at[pl.ds(off, _C)]], bufb, s2)
            cq.wait()
            ck.wait()

            def group_a(g, _):
                rv16 = row2d[ci, pl.ds(g * 16, 16)]
                w16 = jnp.zeros((16,), _F32)
                for k in range(16):
                    e2 = g * 16 + k
                    a = bufa[e2, pl.ds(0, 16)] * bufb[e2, pl.ds(0, 16)]
                    for j in range(1, 8):
                        a = a + (bufa[e2, pl.ds(j * 16, 16)]
                                 * bufb[e2, pl.ds(j * 16, 16)])
                    cs = plsc.cumsum(a)
                    csr = plsc.cumsum(lax.rev(a, (0,)))
                    w = jnp.exp(cs + lax.rev(csr, (0,)) - a)
                    mk = lane == k
                    w16 = jnp.where(mk, w, w16)
                    plsc.addupdate_scatter(wloc, [rv16], w, mask=mk)
                wfull[pl.ds(si * _SUP + ci * _C + g * 16, 16)] = w16
                return 0
            lax.fori_loop(0, _C // 16, group_a, 0)
            return 0
        lax.fori_loop(0, _CPS, chunk_a, 0)
        return 0
    lax.fori_loop(0, nsup, super_a, 0)

    pltpu.sync_copy(wloc, wstage.at[s])

    def half_body(h, _):
        hbase = h * _HB
        hsize = jnp.where(h == 0, _HB, n - _HB)

        def zrow(r, _):
            for j in range(8):
                dchunk[r, pl.ds(j * 16, 16)] = jnp.zeros((16,), _F32)
            return 0
        lax.fori_loop(0, _G, zrow, 0)

        def zcp(k2, _):
            pltpu.sync_copy(dchunk, acc.at[pl.ds(s * _SLAB + k2 * _G, _G)])
            return 0
        lax.fori_loop(0, _SLAB // _G, zcp, 0)
        plsc.subcore_barrier()

        pltpu.sync_copy(wstage.at[:, pl.ds(hbase + (s // 2) * 640, 640)], wred)

        def super_b(si, _):
            load_super(si)

            def chunk_b(ci, _):
                off = ci * _C
                cm = pltpu.async_copy(m_tab.at[cadj.at[pl.ds(off, _C)]],
                                      bufa, s1)
                cm.wait()

                def group_b(g, _):
                    rv16 = row2d[ci, pl.ds(g * 16, 16)]
                    loc = rv16 - hbase
                    ok = (loc >= 0) & (loc < hsize)
                    idxb[pl.ds(g * 16, 16)] = jnp.where(ok, loc, _DUMP + lane)
                    for k in range(16):
                        e2 = g * 16 + k
                        eid = si * _SUP + ci * _C + e2
                        wv = plsc.load_gather(
                            wfull, [jnp.full((16,), eid, _I32)])
                        for j in range(8):
                            bufb[e2, pl.ds(j * 16, 16)] = (
                                wv * bufa[e2, pl.ds(j * 16, 16)])
                    return 0
                lax.fori_loop(0, _C // 16, group_b, 0)
                pltpu.sync_copy(bufb, acc.at[idxb], add=True)
                return 0
            lax.fori_loop(0, _CPS, chunk_b, 0)
            return 0
        lax.fori_loop(0, nsup, super_b, 0)
        plsc.subcore_barrier()

        nvalid = jnp.clip(hsize - s * _SLAB, 0, _SLAB)
        ng = nvalid // _G
        colb = (s % 2) * _SLAB

        def divg(g, _):
            wv = wred[0, pl.ds(colb + g * 16, 16)]
            for k in range(1, _NT):
                wv = wv + wred[k, pl.ds(colb + g * 16, 16)]
            rec = 1.0 / jnp.where(wv == 0.0, 1.0, wv)
            pltpu.sync_copy(acc.at[pl.ds(s * _SLAB + g * _G, _G)], dchunk)
            for j in range(d):
                jf = jnp.full((16,), j, _I32)
                colv = plsc.load_gather(dchunk, [lane, jf])
                plsc.store_scatter(obuf, [lane, jf], colv * rec)
            pltpu.sync_copy(
                obuf, out_xt.at[pl.ds(cn + hbase + s * _SLAB + g * _G, _G)])
            return 0
        lax.fori_loop(0, ng, divg, 0)
        plsc.subcore_barrier()
        return 0
    lax.fori_loop(0, 2, half_body, 0)


def _edge_phase(q_tab, k_tab, m_tab, erow, ecol):
    n = q_tab.shape[0] // 2
    d = q_tab.shape[1]
    ept = erow.shape[0] // _NT
    mesh = plsc.VectorSubcoreMesh(core_axis_name="c", subcore_axis_name="s")
    kern = pl.kernel(
        _edge_body,
        out_type=jax.ShapeDtypeStruct((2 * n, d), _F32),
        mesh=mesh,
        compiler_params=pltpu.CompilerParams(needs_layout_passes=False),
        scratch_types=[
            pltpu.VMEM((_SUP,), _I32),
            pltpu.VMEM((_SUP,), _I32),
            pltpu.VMEM((_SUP,), _I32),
            pltpu.VMEM((_SUP,), _I32),
            pltpu.VMEM((_CPS, _C), _I32),
            pltpu.VMEM((_C,), _I32),
            pltpu.VMEM((_C, d), _F32),
            pltpu.VMEM((_C, d), _F32),
            pltpu.VMEM((ept,), _F32),
            pltpu.VMEM((_WPAD,), _F32),
            pltpu.VMEM((_NT, 640), _F32),
            pltpu.VMEM((_G, d), _F32),
            pltpu.VMEM((_G, d), _F32),
            pltpu.VMEM_SHARED((_ACCR, d), _F32),
            pltpu.VMEM_SHARED((_NT, _WPAD), _F32),
            pltpu.SemaphoreType.DMA,
            pltpu.SemaphoreType.DMA,
        ],
    )
    return kern(q_tab, k_tab, m_tab, erow, ecol)


def kernel(x, t, edge_index, W_x, W_t, Q_alpha_w, Q_alpha_b, K_alpha_w,
           K_alpha_b, Q_beta_w, Q_beta_b, K_beta_w, K_beta_b):
    n = x.shape[0]
    xt = jnp.concatenate([x, t], axis=0)
    qw = jnp.stack([Q_beta_w, Q_alpha_w])
    qb = jnp.stack([Q_beta_b, Q_alpha_b])[:, None, :]
    kw = jnp.stack([K_beta_w, K_alpha_w])
    kb = jnp.stack([K_beta_b, K_alpha_b])[:, None, :]
    ww = jnp.stack([W_x, W_t])
    q_tab, k_tab, m_tab = _tables(xt, qw, qb, kw, kb, ww)
    out_xt = _edge_phase(q_tab, k_tab, m_tab, edge_index[0], edge_index[1])
    return out_xt[:n], out_xt[n:]


# speedup vs baseline: 7.7858x; 1.2506x over previous
import jax
import jax.numpy as jnp
from jax import lax
from jax.experimental import pallas as pl
from jax.experimental.pallas import tpu as pltpu
from jax.experimental.pallas import tpu_sc as plsc

_F32 = jnp.float32
_I32 = jnp.int32


def _tables_body(xt_ref, qw_ref, qb_ref, kw_ref, kb_ref, ww_ref,
                 q_out, k_out, m_out):
    xb = xt_ref[...]
    d = xb.shape[1]
    scale = 1.0 / (d ** 0.5)
    dn = (((1,), (1,)), ((), ()))
    q = lax.dot_general(xb, qw_ref[0], dn, preferred_element_type=_F32)
    q_out[...] = (q + qb_ref[0]) * scale
    k = lax.dot_general(xb, kw_ref[0], dn, preferred_element_type=_F32)
    k_out[...] = k + kb_ref[0]
    m_out[...] = lax.dot_general(xb, ww_ref[0], dn,
                                 preferred_element_type=_F32)


def _tables(xt, qw, qb, kw, kb, ww):
    n2, d = xt.shape
    bn = 2000
    nb = (n2 // 2) // bn
    grid = (2, nb)
    row = pl.BlockSpec((bn, d), lambda i, j: (i * nb + j, 0))
    w_s = pl.BlockSpec((1, d, d), lambda i, j: (i, 0, 0))
    b_s = pl.BlockSpec((1, 1, d), lambda i, j: (i, 0, 0))
    return pl.pallas_call(
        _tables_body,
        grid=grid,
        in_specs=[row, w_s, b_s, w_s, b_s, w_s],
        out_specs=[row, row, row],
        out_shape=[jax.ShapeDtypeStruct((n2, d), _F32)] * 3,
    )(xt, qw, qb, kw, kb, ww)


_NT = 16
_SUP = 2000
_C = 80
_CPS = _SUP // _C
_G = 16
_HB = 3328
_NH = 3
_SLAB = 224
_ACCR = 3600
_DUMP = 3584
_WPAD = 10240


def _edge_body(q_tab, k_tab, m_tab, erow, ecol, out_xt,
               row_f, col_f, radj, cadj, row2d, idxb,
               bufa, bufb, bufa1, bufb1, wfull, wloc, wred, dchunk, obuf,
               iidx, iidxz, acc, wsum_sh, s1, s2, s3, s4):
    n = out_xt.shape[0] // 2
    d = out_xt.shape[1]
    e = erow.shape[0]
    ept = e // _NT
    nsup = ept // _SUP
    c = lax.axis_index("c")
    s = lax.axis_index("s")
    cn = c * n
    ebase = s * ept
    lane = lax.iota(_I32, 16)

    def zw(r, _):
        for j in range(8):
            wloc[r, pl.ds(j * 16, 16)] = jnp.zeros((16,), _F32)
        return 0
    lax.fori_loop(0, _WPAD // 128, zw, 0)

    def zrow(r, _):
        for j in range(8):
            dchunk[r, pl.ds(j * 16, 16)] = jnp.zeros((16,), _F32)
        return 0
    lax.fori_loop(0, _G, zrow, 0)

    iidxz[pl.ds(0, 16)] = jnp.clip(s * 5 + lane, 0, _WPAD // 128 - 1)
    pltpu.sync_copy(dchunk, wsum_sh.at[iidxz])
    def fidx(k, _):
        iidx[pl.ds(k * 16, 16)] = k * 16 + lane
        return 0
    lax.fori_loop(0, 5, fidx, 0)
    plsc.subcore_barrier()

    def load_super(si):
        base = ebase + si * _SUP
        pltpu.sync_copy(erow.at[pl.ds(base, _SUP)], row_f)
        pltpu.sync_copy(ecol.at[pl.ds(base, _SUP)], col_f)

        def adj(k, _):
            rv = row_f[pl.ds(k * 16, 16)]
            cv = col_f[pl.ds(k * 16, 16)]
            radj[pl.ds(k * 16, 16)] = rv + cn
            cadj[pl.ds(k * 16, 16)] = cv + cn
            ci = k // (_C // 16)
            off = (k % (_C // 16)) * 16
            row2d[ci, pl.ds(off, 16)] = rv
            return 0
        lax.fori_loop(0, _SUP // 16, adj, 0)

    def wait_into(dst, sem):
        pltpu.make_async_copy(q_tab.at[radj.at[pl.ds(0, _C)]], dst, sem).wait()

    def fire_a(ci, ba, bb, sa, sb):
        off = ci * _C
        pltpu.async_copy(q_tab.at[radj.at[pl.ds(off, _C)]], ba, sa)
        pltpu.async_copy(k_tab.at[cadj.at[pl.ds(off, _C)]], bb, sb)

    def super_a(si, _):
        load_super(si)

        def proc_a(ci, ba, bb):
            def group_a(g, _):
                rv16 = row2d[ci, pl.ds(g * 16, 16)]
                rhi = jnp.right_shift(rv16, 7)
                rlo = jnp.bitwise_and(rv16, 127)
                w16 = jnp.zeros((16,), _F32)
                for k in range(16):
                    e2 = g * 16 + k
                    a = ba[e2, pl.ds(0, 16)] * bb[e2, pl.ds(0, 16)]
                    for j in range(1, 8):
                        a = a + (ba[e2, pl.ds(j * 16, 16)]
                                 * bb[e2, pl.ds(j * 16, 16)])
                    cs = plsc.cumsum(a)
                    csr = plsc.cumsum(lax.rev(a, (0,)))
                    w = jnp.exp(cs + lax.rev(csr, (0,)) - a)
                    mk = lane == k
                    w16 = jnp.where(mk, w, w16)
                    plsc.addupdate_scatter(wloc, [rhi, rlo], w, mask=mk)
                wfull[pl.ds(si * _SUP + ci * _C + g * 16, 16)] = w16
                return 0
            lax.fori_loop(0, _C // 16, group_a, 0)

        fire_a(0, bufa, bufb, s1, s2)

        def pair_a(i2, _):
            ca = 2 * i2
            fire_a(ca + 1, bufa1, bufb1, s3, s4)
            wait_into(bufa, s1)
            wait_into(bufb, s2)
            proc_a(ca, bufa, bufb)
            fire_a(ca + 2, bufa, bufb, s1, s2)
            wait_into(bufa1, s3)
            wait_into(bufb1, s4)
            proc_a(ca + 1, bufa1, bufb1)
            return 0
        lax.fori_loop(0, _CPS // 2, pair_a, 0)
        wait_into(bufa, s1)
        wait_into(bufb, s2)
        proc_a(_CPS - 1, bufa, bufb)
        return 0
    lax.fori_loop(0, nsup, super_a, 0)

    pltpu.sync_copy(wloc, wsum_sh.at[iidx], add=True)
    plsc.subcore_barrier()
    pltpu.sync_copy(wsum_sh, wred)

    def half_body(h, _):
        hbase = h * _HB
        hsize = jnp.where(h == _NH - 1, n - (_NH - 1) * _HB, _HB)

        def zrow2(r, _):
            for j in range(8):
                dchunk[r, pl.ds(j * 16, 16)] = jnp.zeros((16,), _F32)
            return 0
        lax.fori_loop(0, _G, zrow2, 0)

        def zcp(k2, _):
            pltpu.sync_copy(dchunk, acc.at[pl.ds(s * _SLAB + k2 * _G, _G)])
            return 0
        lax.fori_loop(0, _SLAB // _G, zcp, 0)
        plsc.subcore_barrier()

        def fire_b(ci, ba, sa):
            off = ci * _C
            pltpu.async_copy(m_tab.at[cadj.at[pl.ds(off, _C)]], ba, sa)

        def super_b(si, _):
            load_super(si)

            def proc_b(ci, ba):
                def group_b(g, _):
                    rv16 = row2d[ci, pl.ds(g * 16, 16)]
                    loc = rv16 - hbase
                    ok = (loc >= 0) & (loc < hsize)
                    idxb[pl.ds(g * 16, 16)] = jnp.where(ok, loc, _DUMP + lane)
                    for k in range(16):
                        e2 = g * 16 + k
                        eid = si * _SUP + ci * _C + e2
                        wv = plsc.load_gather(
                            wfull, [jnp.full((16,), eid, _I32)])
                        for j in range(8):
                            bufb[e2, pl.ds(j * 16, 16)] = (
                                wv * ba[e2, pl.ds(j * 16, 16)])
                    return 0
                lax.fori_loop(0, _C // 16, group_b, 0)
                pltpu.sync_copy(bufb, acc.at[idxb], add=True)

            fire_b(0, bufa, s1)

            def pair_b(i2, _):
                cb = 2 * i2
                fire_b(cb + 1, bufa1, s3)
                wait_into(bufa, s1)
                proc_b(cb, bufa)
                fire_b(cb + 2, bufa, s1)
                wait_into(bufa1, s3)
                proc_b(cb + 1, bufa1)
                return 0
            lax.fori_loop(0, _CPS // 2, pair_b, 0)
            wait_into(bufa, s1)
            proc_b(_CPS - 1, bufa)
            return 0
        lax.fori_loop(0, nsup, super_b, 0)
        plsc.subcore_barrier()

        nvalid = jnp.clip(hsize - s * _SLAB, 0, _SLAB)
        ng = nvalid // _G

        def divg(g, _):
            gid = hbase + s * _SLAB + g * _G
            wv = wred[gid // 128, pl.ds(gid % 128, 16)]
            rec = 1.0 / jnp.where(wv == 0.0, 1.0, wv)
            pltpu.sync_copy(acc.at[pl.ds(s * _SLAB + g * _G, _G)], dchunk)
            for j in range(d):
                jf = jnp.full((16,), j, _I32)
                colv = plsc.load_gather(dchunk, [lane, jf])
                plsc.store_scatter(obuf, [lane, jf], colv * rec)
            pltpu.sync_copy(
                obuf, out_xt.at[pl.ds(cn + hbase + s * _SLAB + g * _G, _G)])
            return 0
        lax.fori_loop(0, ng, divg, 0)
        plsc.subcore_barrier()
        return 0
    lax.fori_loop(0, _NH, half_body, 0)


def _edge_phase(q_tab, k_tab, m_tab, erow, ecol):
    n = q_tab.shape[0] // 2
    d = q_tab.shape[1]
    ept = erow.shape[0] // _NT
    mesh = plsc.VectorSubcoreMesh(core_axis_name="c", subcore_axis_name="s")
    kern = pl.kernel(
        _edge_body,
        out_type=jax.ShapeDtypeStruct((2 * n, d), _F32),
        mesh=mesh,
        compiler_params=pltpu.CompilerParams(needs_layout_passes=False),
        scratch_types=[
            pltpu.VMEM((_SUP,), _I32),
            pltpu.VMEM((_SUP,), _I32),
            pltpu.VMEM((_SUP,), _I32),
            pltpu.VMEM((_SUP,), _I32),
            pltpu.VMEM((_CPS, _C), _I32),
            pltpu.VMEM((_C,), _I32),
            pltpu.VMEM((_C, d), _F32),
            pltpu.VMEM((_C, d), _F32),
            pltpu.VMEM((_C, d), _F32),
            pltpu.VMEM((_C, d), _F32),
            pltpu.VMEM((ept,), _F32),
            pltpu.VMEM((_WPAD // 128, d), _F32),
            pltpu.VMEM((_WPAD // 128, d), _F32),
            pltpu.VMEM((_G, d), _F32),
            pltpu.VMEM((_G, d), _F32),
            pltpu.VMEM((80,), _I32),
            pltpu.VMEM((16,), _I32),
            pltpu.VMEM_SHARED((_ACCR, d), _F32),
            pltpu.VMEM_SHARED((_WPAD // 128, d), _F32),
            pltpu.SemaphoreType.DMA,
            pltpu.SemaphoreType.DMA,
            pltpu.SemaphoreType.DMA,
            pltpu.SemaphoreType.DMA,
        ],
    )
    return kern(q_tab, k_tab, m_tab, erow, ecol)


def kernel(x, t, edge_index, W_x, W_t, Q_alpha_w, Q_alpha_b, K_alpha_w,
           K_alpha_b, Q_beta_w, Q_beta_b, K_beta_w, K_beta_b):
    n = x.shape[0]
    xt = jnp.concatenate([x, t], axis=0)
    qw = jnp.stack([Q_beta_w, Q_alpha_w])
    qb = jnp.stack([Q_beta_b, Q_alpha_b])[:, None, :]
    kw = jnp.stack([K_beta_w, K_alpha_w])
    kb = jnp.stack([K_beta_b, K_alpha_b])[:, None, :]
    ww = jnp.stack([W_x, W_t])
    q_tab, k_tab, m_tab = _tables(xt, qw, qb, kw, kb, ww)
    out_xt = _edge_phase(q_tab, k_tab, m_tab, edge_index[0], edge_index[1])
    return out_xt[:n], out_xt[n:]
